# baseline (speedup 1.0000x reference)
import functools

import jax
import jax.numpy as jnp
from jax import lax
from jax.experimental import pallas as pl
from jax.experimental.pallas import tpu as pltpu
from jax.experimental.pallas import tpu_sc as plsc

_B, _C, _H, _D, _F, _G = 8, 2048, 16, 64, 64, 1024
_LANES = _B * _H

_STAGES = 11
_DD = []
_SH = []
for _stage in range(_STAGES):
    for _sub in range(_stage, -1, -1):
        _DD.append(1 << _sub)
        _SH.append(_stage + 1)
_NPASS = len(_DD)


def _sort_body(dd_ref, sh_ref, score_ref, key_out_ref, idx_out_ref, x2, i2):
    pos = lax.broadcasted_iota(jnp.int32, (_C, _LANES), 0)
    x2[0:_C] = score_ref[...]
    i2[0:_C] = pos

    def body(t, carry):
        sh = sh_ref[0, t]
        sub = dd_ref[0, t]
        asc = ((pos >> sh) & 1) == 0
        a = x2[0:_C]
        ai = i2[0:_C]
        for s in range(_STAGES):
            @pl.when(sub == s)
            def _(s=s, a=a, ai=ai):
                d = 1 << s
                lower = (pos & d) == 0
                pk = jnp.where(lower, jnp.roll(a, -d, axis=0),
                               jnp.roll(a, d, axis=0))
                pi = jnp.where(lower, jnp.roll(ai, -d, axis=0),
                               jnp.roll(ai, d, axis=0))
                less = (a > pk) | ((a == pk) & (ai < pi))
                take_self = less == (asc == lower)
                x2[0:_C] = jnp.where(take_self, a, pk)
                i2[0:_C] = jnp.where(take_self, ai, pi)
        return carry

    lax.fori_loop(0, _NPASS, body, 0)
    key_out_ref[...] = x2[0:_G]
    idx_out_ref[...] = i2[0:_G]


_sort_call_inner = pl.pallas_call(
    _sort_body,
    in_specs=[
        pl.BlockSpec(memory_space=pltpu.MemorySpace.SMEM),
        pl.BlockSpec(memory_space=pltpu.MemorySpace.SMEM),
        pl.BlockSpec(memory_space=pltpu.MemorySpace.VMEM),
    ],
    out_shape=(
        jax.ShapeDtypeStruct((_G, _LANES), jnp.float32),
        jax.ShapeDtypeStruct((_G, _LANES), jnp.int32),
    ),
    scratch_shapes=[
        pltpu.VMEM((_C, _LANES), jnp.float32),
        pltpu.VMEM((_C, _LANES), jnp.int32),
    ],
)


def _sort_call(score_t):
    sub = jnp.asarray([d.bit_length() - 1 for d in _DD],
                      dtype=jnp.int32).reshape(1, _NPASS)
    sh = jnp.asarray(_SH, dtype=jnp.int32).reshape(1, _NPASS)
    return _sort_call_inner(sub, sh, score_t)

_NW = 32
_PPW = _LANES // _NW
_TR = _D // 8
_TC = _C // 128
_TG = _G // 128


_NUNITS = _PPW * _TR


def _gather_body(cidx_hbm, k_hbm, v_hbm, fk_hbm, ko, vo, fko,
                 cidx_v, slk0, slv0, slf0, slk1, slv1, slf1,
                 obk, obv, obf,
                 sik0, siv0, sif0, sik1, siv1, sif1, sok, sov, sof):
    cid = lax.axis_index("c")
    sid = lax.axis_index("s")
    wid = sid * 2 + cid
    pltpu.sync_copy(cidx_hbm.at[wid], cidx_v)
    slabs = ((slk0, slv0, slf0), (slk1, slv1, slf1))
    obufs = (obk, obv, obf)
    isems = ((sik0, siv0, sif0), (sik1, siv1, sif1))
    osems = (sok, sov, sof)

    def _pa(u):
        p = u // _TR
        return wid * _PPW + p, u % _TR, p

    def fire_in(u, bset):
        pair, a, _ = _pa(u)
        for t, src in enumerate((k_hbm, v_hbm, fk_hbm)):
            pltpu.async_copy(src.at[pair, a], slabs[bset][t], isems[bset][t])

    def drain_in(u, bset):
        pair, a, _ = _pa(u)
        for t, src in enumerate((k_hbm, v_hbm, fk_hbm)):
            pltpu.make_async_copy(src.at[pair, a], slabs[bset][t],
                                  isems[bset][t]).wait()

    def drain_out(u):
        pair, a, _ = _pa(u)
        for t, dst in enumerate((ko, vo, fko)):
            pltpu.make_async_copy(obufs[t], dst.at[pair, a], osems[t]).wait()

    fire_in(0, 0)

    def outer(s, carry):
        for b in range(2):
            u = 2 * s + b
            nxt = u + 1

            @pl.when(nxt < _NUNITS)
            def _():
                fire_in(nxt, (b + 1) % 2)

            pair, a, p = _pa(u)

            @pl.when(u > 0)
            def _():
                drain_out(u - 1)

            drain_in(u, b)
            for t, dst in enumerate((ko, vo, fko)):
                slab = slabs[b][t]
                obuf = obufs[t]

                def jgroup(jg, c2, slab=slab, obuf=obuf, p=p):
                    for jj in range(8):
                        lo16 = jj * 16
                        idxv = cidx_v[p * _TG + jg, pl.ds(lo16, 16)]
                        row = lax.shift_right_logical(idxv, 7) * 8
                        lo = lax.bitwise_and(idxv, 127)
                        for r in range(8):
                            g = plsc.load_gather(slab, [row + r, lo])
                            obuf[jg * 8 + r, pl.ds(lo16, 16)] = g
                    return c2

                lax.fori_loop(0, _TG, jgroup, 0)
                pltpu.async_copy(obuf, dst.at[pair, a], osems[t])
        return carry

    lax.fori_loop(0, _NUNITS // 2, outer, 0)
    drain_out(_NUNITS - 1)


def _gather_call(cidx3, ktab, vtab, fktab):
    mesh = plsc.VectorSubcoreMesh(core_axis_name="c", subcore_axis_name="s")
    out_sds = jax.ShapeDtypeStruct((_LANES, _TR, _TG * 8, 128), jnp.float32)
    slab_t = pltpu.VMEM((_TC * 8, 128), jnp.float32)
    obuf_t = pltpu.VMEM((_TG * 8, 128), jnp.float32)
    f = functools.partial(
        pl.kernel,
        mesh=mesh,
        out_type=(out_sds, out_sds, out_sds),
        scratch_types=[
            pltpu.VMEM((_PPW * _TG, 128), jnp.int32),
            slab_t, slab_t, slab_t, slab_t, slab_t, slab_t,
            obuf_t, obuf_t, obuf_t,
            pltpu.SemaphoreType.DMA, pltpu.SemaphoreType.DMA,
            pltpu.SemaphoreType.DMA, pltpu.SemaphoreType.DMA,
            pltpu.SemaphoreType.DMA, pltpu.SemaphoreType.DMA,
            pltpu.SemaphoreType.DMA, pltpu.SemaphoreType.DMA,
            pltpu.SemaphoreType.DMA,
        ],
        compiler_params=pltpu.CompilerParams(needs_layout_passes=False),
    )(_gather_body)
    return f(cidx3, ktab, vtab, fktab)

_HG = 4


def _einsum_full_body(fk_ref, v_ref, m_ref, s_ref):
    fk = fk_ref[0]
    v = v_ref[0]
    dn = (((1,), (1,)), ((), ()))
    m_ref[0] = lax.dot_general(fk, v, dn, preferred_element_type=jnp.float32)
    ones_c = jnp.ones((1, _C), dtype=jnp.float32)
    s = lax.dot_general(ones_c, fk, dn, preferred_element_type=jnp.float32)
    s_ref[0] = jnp.broadcast_to(s, (8, _HG * _F))


_einsum_full_call = pl.pallas_call(
    _einsum_full_body,
    grid=(_B, _H // _HG),
    in_specs=[
        pl.BlockSpec((1, _HG * _F, _C), lambda b, g: (b, g, 0)),
        pl.BlockSpec((1, _HG * _D, _C), lambda b, g: (b, g, 0)),
    ],
    out_specs=(
        pl.BlockSpec((1, _HG * _F, _HG * _D),
                     lambda b, g: (b * (_H // _HG) + g, 0, 0)),
        pl.BlockSpec((1, 8, _HG * _F), lambda b, g: (b * (_H // _HG) + g, 0, 0)),
    ),
    out_shape=(
        jax.ShapeDtypeStruct((_B * (_H // _HG), _HG * _F, _HG * _D), jnp.float32),
        jax.ShapeDtypeStruct((_B * (_H // _HG), 8, _HG * _F), jnp.float32),
    ),
)


def _einsum_top_body(m_ref, sf_ref, fkt_ref, vt_ref, h_ref, s_ref):
    fkt = fkt_ref[0]
    vt = vt_ref[0]
    dn = (((1,), (1,)), ((), ()))
    m = m_ref[0] - lax.dot_general(fkt, vt, dn,
                                   preferred_element_type=jnp.float32)
    ones_g = jnp.ones((1, _G), dtype=jnp.float32)
    st = lax.dot_general(ones_g, fkt, dn, preferred_element_type=jnp.float32)
    s_ref[0] = sf_ref[0] - jnp.broadcast_to(st, (8, _HG * _F))
    for j in range(_HG):
        h_ref[0, j] = m[j * _F:(j + 1) * _F, j * _D:(j + 1) * _D]


_einsum_top_call = pl.pallas_call(
    _einsum_top_body,
    grid=(_B, _H // _HG),
    in_specs=[
        pl.BlockSpec((1, _HG * _F, _HG * _D),
                     lambda b, g: (b * (_H // _HG) + g, 0, 0)),
        pl.BlockSpec((1, 8, _HG * _F), lambda b, g: (b * (_H // _HG) + g, 0, 0)),
        pl.BlockSpec((1, _HG * _F, _G), lambda b, g: (b, g, 0)),
        pl.BlockSpec((1, _HG * _D, _G), lambda b, g: (b, g, 0)),
    ],
    out_specs=(
        pl.BlockSpec((1, _HG, _F, _D), lambda b, g: (b, g, 0, 0)),
        pl.BlockSpec((1, 8, _HG * _F), lambda b, g: (b * (_H // _HG) + g, 0, 0)),
    ),
    out_shape=(
        jax.ShapeDtypeStruct((_B, _H, _F, _D), jnp.float32),
        jax.ShapeDtypeStruct((_B * (_H // _HG), 8, _HG * _F), jnp.float32),
    ),
)


def _to_tiled(x):
    return (x.transpose(0, 2, 3, 1)
            .reshape(_LANES, _TR, 8, _TC, 128)
            .transpose(0, 1, 3, 2, 4)
            .reshape(_LANES, _TR, _TC * 8, 128))


def _from_tiled(x):
    return (x.reshape(_LANES, _TR, _TG, 8, 128)
            .transpose(0, 1, 3, 2, 4)
            .reshape(_B, _H, _D, _G)
            .transpose(0, 3, 1, 2))


def kernel(k_c, v_c, fk_c, score_c):
    score_t = score_c.transpose(1, 0, 2).reshape(_C, _LANES)
    keys_top, cidx_raw = _sort_call(score_t)
    heap_score = keys_top.reshape(_G, _B, _H).transpose(1, 0, 2)
    cidx3 = cidx_raw.transpose(1, 0).reshape(_NW, _PPW * _TG, 128)

    ktab = _to_tiled(k_c)
    vtab = _to_tiled(v_c)
    fktab = _to_tiled(fk_c)
    ko, vo, fko = _gather_call(cidx3, ktab, vtab, fktab)

    def _top_flat(x):
        return (x.reshape(_LANES, _TR, _TG, 8, 128)
                .transpose(0, 1, 3, 2, 4)
                .reshape(_B, _H * _D, _G))

    m_full, s_full = _einsum_full_call(
        fk_c.transpose(0, 2, 3, 1).reshape(_B, _H * _F, _C),
        v_c.transpose(0, 2, 3, 1).reshape(_B, _H * _D, _C),
    )
    h_sum, s_sum = _einsum_top_call(
        m_full, s_full, _top_flat(fko), _top_flat(vo),
    )
    return (
        _from_tiled(ko),
        _from_tiled(vo),
        _from_tiled(fko),
        heap_score,
        h_sum,
        s_sum[:, 0, :].reshape(_B, _H, _F),
    )


# speedup vs baseline: 1.0266x; 1.0266x over previous
import functools

import jax
import jax.numpy as jnp
from jax import lax
from jax.experimental import pallas as pl
from jax.experimental.pallas import tpu as pltpu
from jax.experimental.pallas import tpu_sc as plsc

_B, _C, _H, _D, _F, _G = 8, 2048, 16, 64, 64, 1024
_LANES = _B * _H

_STAGES = 11
_DD = []
_SH = []
for _stage in range(_STAGES):
    for _sub in range(_stage, -1, -1):
        _DD.append(1 << _sub)
        _SH.append(_stage + 1)
_NPASS = len(_DD)


_OFF = _C // 2


def _sort_body(dd_ref, sh_ref, score_ref, key_out_ref, idx_out_ref, x2, i2):
    pos = lax.broadcasted_iota(jnp.int32, (_C, _LANES), 0)
    x2[_OFF:_OFF + _C] = score_ref[...]
    i2[_OFF:_OFF + _C] = pos

    def body(t, carry):
        d = dd_ref[0, t]
        sh = sh_ref[0, t]
        lower = (pos & d) == 0
        asc = ((pos >> sh) & 1) == 0
        a = x2[_OFF:_OFF + _C]
        ai = i2[_OFF:_OFF + _C]
        pk = jnp.where(lower, x2[pl.ds(_OFF + d, _C)], x2[pl.ds(_OFF - d, _C)])
        pi = jnp.where(lower, i2[pl.ds(_OFF + d, _C)], i2[pl.ds(_OFF - d, _C)])
        less = (a > pk) | ((a == pk) & (ai < pi))
        take_self = less == (asc == lower)
        x2[_OFF:_OFF + _C] = jnp.where(take_self, a, pk)
        i2[_OFF:_OFF + _C] = jnp.where(take_self, ai, pi)
        return carry

    lax.fori_loop(0, _NPASS, body, 0)
    key_out_ref[...] = x2[_OFF:_OFF + _G]
    idx_out_ref[...] = i2[_OFF:_OFF + _G]


_sort_call_inner = pl.pallas_call(
    _sort_body,
    in_specs=[
        pl.BlockSpec(memory_space=pltpu.MemorySpace.SMEM),
        pl.BlockSpec(memory_space=pltpu.MemorySpace.SMEM),
        pl.BlockSpec(memory_space=pltpu.MemorySpace.VMEM),
    ],
    out_shape=(
        jax.ShapeDtypeStruct((_G, _LANES), jnp.float32),
        jax.ShapeDtypeStruct((_G, _LANES), jnp.int32),
    ),
    scratch_shapes=[
        pltpu.VMEM((2 * _C, _LANES), jnp.float32),
        pltpu.VMEM((2 * _C, _LANES), jnp.int32),
    ],
)


def _sort_call(score_t):
    dd = jnp.asarray(_DD, dtype=jnp.int32).reshape(1, _NPASS)
    sh = jnp.asarray(_SH, dtype=jnp.int32).reshape(1, _NPASS)
    return _sort_call_inner(dd, sh, score_t)

_NW = 32
_PPW = _LANES // _NW
_TR = _D // 8
_TC = _C // 128
_TG = _G // 128


_NUNITS = _PPW * _TR


def _gather_body(cidx_hbm, k_hbm, v_hbm, fk_hbm, ko, vo, fko,
                 cidx_v, slk0, slv0, slf0, slk1, slv1, slf1,
                 obk, obv, obf,
                 sik0, siv0, sif0, sik1, siv1, sif1, sok, sov, sof):
    cid = lax.axis_index("c")
    sid = lax.axis_index("s")
    wid = sid * 2 + cid
    pltpu.sync_copy(cidx_hbm.at[wid], cidx_v)
    slabs = ((slk0, slv0, slf0), (slk1, slv1, slf1))
    obufs = (obk, obv, obf)
    isems = ((sik0, siv0, sif0), (sik1, siv1, sif1))
    osems = (sok, sov, sof)

    def _pa(u):
        p = u // _TR
        return wid * _PPW + p, u % _TR, p

    def fire_in(u, bset):
        pair, a, _ = _pa(u)
        for t, src in enumerate((k_hbm, v_hbm, fk_hbm)):
            pltpu.async_copy(src.at[pair, a], slabs[bset][t], isems[bset][t])

    def drain_in(u, bset):
        pair, a, _ = _pa(u)
        for t, src in enumerate((k_hbm, v_hbm, fk_hbm)):
            pltpu.make_async_copy(src.at[pair, a], slabs[bset][t],
                                  isems[bset][t]).wait()

    def drain_out(u):
        pair, a, _ = _pa(u)
        for t, dst in enumerate((ko, vo, fko)):
            pltpu.make_async_copy(obufs[t], dst.at[pair, a], osems[t]).wait()

    fire_in(0, 0)

    def outer(s, carry):
        for b in range(2):
            u = 2 * s + b
            nxt = u + 1

            @pl.when(nxt < _NUNITS)
            def _():
                fire_in(nxt, (b + 1) % 2)

            pair, a, p = _pa(u)

            @pl.when(u > 0)
            def _():
                drain_out(u - 1)

            drain_in(u, b)
            for t, dst in enumerate((ko, vo, fko)):
                slab = slabs[b][t]
                obuf = obufs[t]

                def jgroup(jg, c2, slab=slab, obuf=obuf, p=p):
                    for jj in range(8):
                        lo16 = jj * 16
                        idxv = cidx_v[p * _TG + jg, pl.ds(lo16, 16)]
                        row = lax.shift_right_logical(idxv, 7) * 8
                        lo = lax.bitwise_and(idxv, 127)
                        for r in range(8):
                            g = plsc.load_gather(slab, [row + r, lo])
                            obuf[jg * 8 + r, pl.ds(lo16, 16)] = g
                    return c2

                lax.fori_loop(0, _TG, jgroup, 0)
                pltpu.async_copy(obuf, dst.at[pair, a], osems[t])
        return carry

    lax.fori_loop(0, _NUNITS // 2, outer, 0)
    drain_out(_NUNITS - 1)


def _gather_call(cidx3, ktab, vtab, fktab):
    mesh = plsc.VectorSubcoreMesh(core_axis_name="c", subcore_axis_name="s")
    out_sds = jax.ShapeDtypeStruct((_LANES, _TR, _TG * 8, 128), jnp.float32)
    slab_t = pltpu.VMEM((_TC * 8, 128), jnp.float32)
    obuf_t = pltpu.VMEM((_TG * 8, 128), jnp.float32)
    f = functools.partial(
        pl.kernel,
        mesh=mesh,
        out_type=(out_sds, out_sds, out_sds),
        scratch_types=[
            pltpu.VMEM((_PPW * _TG, 128), jnp.int32),
            slab_t, slab_t, slab_t, slab_t, slab_t, slab_t,
            obuf_t, obuf_t, obuf_t,
            pltpu.SemaphoreType.DMA, pltpu.SemaphoreType.DMA,
            pltpu.SemaphoreType.DMA, pltpu.SemaphoreType.DMA,
            pltpu.SemaphoreType.DMA, pltpu.SemaphoreType.DMA,
            pltpu.SemaphoreType.DMA, pltpu.SemaphoreType.DMA,
            pltpu.SemaphoreType.DMA,
        ],
        compiler_params=pltpu.CompilerParams(needs_layout_passes=False),
    )(_gather_body)
    return f(cidx3, ktab, vtab, fktab)

_HG = 4


def _einsum_full_body(fk_ref, v_ref, m_ref, s_ref):
    fk = fk_ref[0]
    v = v_ref[0]
    dn = (((1,), (1,)), ((), ()))
    m_ref[0] = lax.dot_general(fk, v, dn, preferred_element_type=jnp.float32)
    ones_c = jnp.ones((1, _C), dtype=jnp.float32)
    s = lax.dot_general(ones_c, fk, dn, preferred_element_type=jnp.float32)
    s_ref[0] = jnp.broadcast_to(s, (8, _HG * _F))


_einsum_full_call = pl.pallas_call(
    _einsum_full_body,
    grid=(_B, _H // _HG),
    in_specs=[
        pl.BlockSpec((1, _HG * _F, _C), lambda b, g: (b, g, 0)),
        pl.BlockSpec((1, _HG * _D, _C), lambda b, g: (b, g, 0)),
    ],
    out_specs=(
        pl.BlockSpec((1, _HG * _F, _HG * _D),
                     lambda b, g: (b * (_H // _HG) + g, 0, 0)),
        pl.BlockSpec((1, 8, _HG * _F), lambda b, g: (b * (_H // _HG) + g, 0, 0)),
    ),
    out_shape=(
        jax.ShapeDtypeStruct((_B * (_H // _HG), _HG * _F, _HG * _D), jnp.float32),
        jax.ShapeDtypeStruct((_B * (_H // _HG), 8, _HG * _F), jnp.float32),
    ),
)


def _einsum_top_body(m_ref, sf_ref, fkt_ref, vt_ref, h_ref, s_ref):
    fkt = fkt_ref[0]
    vt = vt_ref[0]
    dn = (((1,), (1,)), ((), ()))
    m = m_ref[0] - lax.dot_general(fkt, vt, dn,
                                   preferred_element_type=jnp.float32)
    ones_g = jnp.ones((1, _G), dtype=jnp.float32)
    st = lax.dot_general(ones_g, fkt, dn, preferred_element_type=jnp.float32)
    s_ref[0] = sf_ref[0] - jnp.broadcast_to(st, (8, _HG * _F))
    for j in range(_HG):
        h_ref[0, j] = m[j * _F:(j + 1) * _F, j * _D:(j + 1) * _D]


_einsum_top_call = pl.pallas_call(
    _einsum_top_body,
    grid=(_B, _H // _HG),
    in_specs=[
        pl.BlockSpec((1, _HG * _F, _HG * _D),
                     lambda b, g: (b * (_H // _HG) + g, 0, 0)),
        pl.BlockSpec((1, 8, _HG * _F), lambda b, g: (b * (_H // _HG) + g, 0, 0)),
        pl.BlockSpec((1, _HG * _F, _G), lambda b, g: (b, g, 0)),
        pl.BlockSpec((1, _HG * _D, _G), lambda b, g: (b, g, 0)),
    ],
    out_specs=(
        pl.BlockSpec((1, _HG, _F, _D), lambda b, g: (b, g, 0, 0)),
        pl.BlockSpec((1, 8, _HG * _F), lambda b, g: (b * (_H // _HG) + g, 0, 0)),
    ),
    out_shape=(
        jax.ShapeDtypeStruct((_B, _H, _F, _D), jnp.float32),
        jax.ShapeDtypeStruct((_B * (_H // _HG), 8, _HG * _F), jnp.float32),
    ),
)


def _to_tiled(x):
    return (x.transpose(0, 2, 3, 1)
            .reshape(_LANES, _TR, 8, _TC, 128)
            .transpose(0, 1, 3, 2, 4)
            .reshape(_LANES, _TR, _TC * 8, 128))


def _from_tiled(x):
    return (x.reshape(_LANES, _TR, _TG, 8, 128)
            .transpose(0, 1, 3, 2, 4)
            .reshape(_B, _H, _D, _G)
            .transpose(0, 3, 1, 2))


def kernel(k_c, v_c, fk_c, score_c):
    score_t = score_c.transpose(1, 0, 2).reshape(_C, _LANES)
    keys_top, cidx_raw = _sort_call(score_t)
    heap_score = keys_top.reshape(_G, _B, _H).transpose(1, 0, 2)
    cidx3 = cidx_raw.transpose(1, 0).reshape(_NW, _PPW * _TG, 128)

    ktab = _to_tiled(k_c)
    vtab = _to_tiled(v_c)
    fktab = _to_tiled(fk_c)
    ko, vo, fko = _gather_call(cidx3, ktab, vtab, fktab)

    def _top_flat(x):
        return (x.reshape(_LANES, _TR, _TG, 8, 128)
                .transpose(0, 1, 3, 2, 4)
                .reshape(_B, _H * _D, _G))

    m_full, s_full = _einsum_full_call(
        fk_c.transpose(0, 2, 3, 1).reshape(_B, _H * _F, _C),
        v_c.transpose(0, 2, 3, 1).reshape(_B, _H * _D, _C),
    )
    h_sum, s_sum = _einsum_top_call(
        m_full, s_full, _top_flat(fko), _top_flat(vo),
    )
    return (
        _from_tiled(ko),
        _from_tiled(vo),
        _from_tiled(fko),
        heap_score,
        h_sum,
        s_sum[:, 0, :].reshape(_B, _H, _F),
    )
